# Initial kernel scaffold; baseline (speedup 1.0000x reference)
import jax
import jax.numpy as jnp
from jax.experimental import pallas as pl


def kernel(inputs, embedding_table):
    raise NotImplementedError("write your pallas kernel here")


# speedup vs baseline: 1.1086x; 1.1086x over previous
import functools

import jax
import jax.numpy as jnp
from jax import lax
from jax.experimental import pallas as pl
from jax.experimental.pallas import tpu as pltpu
from jax.experimental.pallas import tpu_sc as plsc

_info = plsc.get_sparse_core_info()
_NC = _info.num_cores
_NS = _info.num_subcores
_NW = _NC * _NS


@functools.lru_cache(maxsize=None)
def _make_gather(num_rows, dim, batch):
    assert batch % _NW == 0
    b_per_w = batch // _NW
    chunk = 1600
    while b_per_w % chunk != 0:
        chunk //= 2
    n_chunks = b_per_w // chunk
    assert chunk % 8 == 0

    mesh = plsc.VectorSubcoreMesh(core_axis_name="c", subcore_axis_name="s")

    @functools.partial(
        pl.kernel,
        out_type=jax.ShapeDtypeStruct((batch, dim), jnp.float32),
        mesh=mesh,
        compiler_params=pltpu.CompilerParams(use_tc_tiling_on_sc=False),
        scratch_types=[
            pltpu.VMEM((chunk,), jnp.int32),
            pltpu.VMEM((chunk,), jnp.int32),
            pltpu.VMEM((chunk, dim), jnp.float32),
            pltpu.VMEM((chunk, dim), jnp.float32),
            pltpu.SemaphoreType.DMA,
            pltpu.SemaphoreType.DMA,
            pltpu.SemaphoreType.DMA,
            pltpu.SemaphoreType.DMA,
        ],
    )
    def grab(table_hbm, idx_hbm, out_hbm, idx_v0, idx_v1, rows_v0, rows_v1,
             g0, g1, s0, s1):
        wid = lax.axis_index("s") * _NC + lax.axis_index("c")
        base = wid * b_per_w
        idx_bufs = (idx_v0, idx_v1)
        row_bufs = (rows_v0, rows_v1)
        gsems = (g0, g1)
        ssems = (s0, s1)
        stores = [None, None]
        prev = None
        for i in range(n_chunks):
            b = i % 2
            if stores[b] is not None:
                stores[b].wait()
            pltpu.sync_copy(idx_hbm.at[pl.ds(base + i * chunk, chunk)],
                            idx_bufs[b])
            g = pltpu.async_copy(table_hbm.at[idx_bufs[b]], row_bufs[b],
                                 gsems[b])
            if prev is not None:
                pb, pg = prev
                pg.wait()
                stores[pb] = pltpu.async_copy(
                    row_bufs[pb],
                    out_hbm.at[pl.ds(base + (i - 1) * chunk, chunk)],
                    ssems[pb])
            prev = (b, g)
        pb, pg = prev
        pg.wait()
        stores[pb] = pltpu.async_copy(
            row_bufs[pb],
            out_hbm.at[pl.ds(base + (n_chunks - 1) * chunk, chunk)],
            ssems[pb])
        for s in stores:
            if s is not None:
                s.wait()

    return grab


def kernel(inputs, embedding_table):
    batch, seq = inputs.shape
    num_rows, dim = embedding_table.shape
    idx = inputs.reshape(-1).astype(jnp.int32)
    grab = _make_gather(num_rows, dim, batch * seq)
    out = grab(embedding_table, idx)
    return out.reshape(batch, seq, dim)


# speedup vs baseline: 1.6035x; 1.4464x over previous
import functools

import jax
import jax.numpy as jnp
from jax import lax
from jax.experimental import pallas as pl
from jax.experimental.pallas import tpu as pltpu
from jax.experimental.pallas import tpu_sc as plsc

_info = plsc.get_sparse_core_info()
_NC = _info.num_cores
_NS = _info.num_subcores
_NW = _NC * _NS


@functools.lru_cache(maxsize=None)
def _make_gather(num_rows, dim, batch, seq):
    assert dim == 32 and batch % 512 == 0
    chunk = 512
    chunks_per_s = batch // chunk
    n_units = seq * chunks_per_s
    assert n_units % _NW == 0
    units_per_w = n_units // _NW
    assert units_per_w >= 4 and units_per_w % 2 == 0
    bt_per_chunk = chunk // 128
    n_dt = dim // 8

    mesh = plsc.VectorSubcoreMesh(core_axis_name="c", subcore_axis_name="s")

    @functools.partial(
        pl.kernel,
        out_type=jax.ShapeDtypeStruct(
            (seq, n_dt, batch // 128, 8, 128), jnp.float32),
        mesh=mesh,
        compiler_params=pltpu.CompilerParams(
            use_tc_tiling_on_sc=False, needs_layout_passes=False),
        scratch_types=[
            pltpu.VMEM((chunk,), jnp.int32),
            pltpu.VMEM((chunk,), jnp.int32),
            pltpu.VMEM((chunk, dim), jnp.float32),
            pltpu.VMEM((chunk, dim), jnp.float32),
            pltpu.VMEM((n_dt, bt_per_chunk, 8, 128), jnp.float32),
            pltpu.VMEM((n_dt, bt_per_chunk, 8, 128), jnp.float32),
            pltpu.SemaphoreType.DMA,
            pltpu.SemaphoreType.DMA,
            pltpu.SemaphoreType.DMA,
            pltpu.SemaphoreType.DMA,
        ],
    )
    def grab(table_hbm, idx_hbm, out_hbm, idx_v0, idx_v1, rows_v0, rows_v1,
             tr_v0, tr_v1, g0, g1, s0, s1):
        wid = lax.axis_index("s") * _NC + lax.axis_index("c")
        base_u = wid * units_per_w
        idx_bufs = (idx_v0, idx_v1)
        row_bufs = (rows_v0, rows_v1)
        tr_bufs = (tr_v0, tr_v1)
        gsems = (g0, g1)
        ssems = (s0, s1)
        lane = lax.iota(jnp.int32, 16)

        def transpose_chunk(rows, tr):
            def blk_body(blk, _):
                row_idx = blk * 16 + lane
                bt = blk >> 3
                bi0 = (blk & 7) * 16
                for dt in range(n_dt):
                    for di in range(8):
                        col_idx = jnp.full((16,), dt * 8 + di, jnp.int32)
                        v = plsc.load_gather(rows, [row_idx, col_idx])
                        tr[dt, bt, di, pl.ds(bi0, 16)] = v
                return ()

            lax.fori_loop(0, chunk // 16, blk_body, (), unroll=False)

        def out_slices(b, u):
            g = base_u + u
            s_idx = g // chunks_per_s
            bt0 = (g % chunks_per_s) * bt_per_chunk
            for dt in range(n_dt):
                yield (tr_bufs[b].at[dt],
                       out_hbm.at[s_idx, dt, pl.ds(bt0, bt_per_chunk)])

        def issue_gather(b, u):
            pltpu.sync_copy(
                idx_hbm.at[pl.ds((base_u + u) * chunk, chunk)], idx_bufs[b])
            pltpu.async_copy(table_hbm.at[idx_bufs[b]], row_bufs[b], gsems[b])

        def wait_gather(b):
            pltpu.make_async_copy(
                table_hbm.at[idx_bufs[b]], row_bufs[b], gsems[b]).wait()

        def issue_stores(b, u):
            for src, dst in out_slices(b, u):
                pltpu.async_copy(src, dst, ssems[b])

        def wait_stores(b, u):
            for src, dst in out_slices(b, u):
                pltpu.make_async_copy(src, dst, ssems[b]).wait()

        def process_unit(b, u, *, wait_store_u=None, next_u=None):
            if wait_store_u is not None:
                wait_stores(b, wait_store_u)
            wait_gather(b)
            transpose_chunk(row_bufs[b], tr_bufs[b])
            issue_stores(b, u)
            if next_u is not None:
                issue_gather(b, next_u)

        issue_gather(0, 0)
        issue_gather(1, 1)
        for b in (0, 1):
            process_unit(b, b, next_u=b + 2)

        def body(k, _):
            for b in (0, 1):
                u = 2 * k + b
                process_unit(b, u, wait_store_u=u - 2, next_u=u + 2)
            return ()

        lax.fori_loop(1, units_per_w // 2 - 1, body, ())

        for b in (0, 1):
            u = units_per_w - 2 + b
            process_unit(b, u, wait_store_u=u - 2)
        for b in (0, 1):
            wait_stores(b, units_per_w - 2 + b)

    return grab


def kernel(inputs, embedding_table):
    batch, seq = inputs.shape
    num_rows, dim = embedding_table.shape
    idx = inputs.T.reshape(-1).astype(jnp.int32)
    grab = _make_gather(num_rows, dim, batch, seq)
    out5 = grab(embedding_table, idx)
    return out5.transpose(2, 4, 0, 1, 3).reshape(batch, seq, dim)


# speedup vs baseline: 2.0072x; 1.2518x over previous
import functools

import jax
import jax.numpy as jnp
from jax import lax
from jax.experimental import pallas as pl
from jax.experimental.pallas import tpu as pltpu
from jax.experimental.pallas import tpu_sc as plsc

_info = plsc.get_sparse_core_info()
_NC = _info.num_cores
_NS = _info.num_subcores
_NW = _NC * _NS


@functools.lru_cache(maxsize=None)
def _make_gather(num_rows, dim, batch, seq):
    assert dim == 32 and batch % 512 == 0
    chunk = 512
    chunks_per_s = batch // chunk
    n_units = seq * chunks_per_s
    assert n_units % _NW == 0
    units_per_w = n_units // _NW
    assert units_per_w >= 4 and units_per_w % 2 == 0
    bt_per_chunk = chunk // 128
    n_dt = dim // 8

    mesh = plsc.VectorSubcoreMesh(core_axis_name="c", subcore_axis_name="s")

    @functools.partial(
        pl.kernel,
        out_type=jax.ShapeDtypeStruct(
            (seq, n_dt, batch // 128, 8, 128), jnp.float32),
        mesh=mesh,
        compiler_params=pltpu.CompilerParams(
            use_tc_tiling_on_sc=False, needs_layout_passes=False),
        scratch_types=[
            pltpu.VMEM((chunk,), jnp.int32),
            pltpu.VMEM((chunk,), jnp.int32),
            pltpu.VMEM((chunk, dim), jnp.float32),
            pltpu.VMEM((chunk, dim), jnp.float32),
            pltpu.VMEM((n_dt, bt_per_chunk, 8, 128), jnp.float32),
            pltpu.VMEM((n_dt, bt_per_chunk, 8, 128), jnp.float32),
            pltpu.SemaphoreType.DMA,
            pltpu.SemaphoreType.DMA,
            pltpu.SemaphoreType.DMA,
            pltpu.SemaphoreType.DMA,
        ],
    )
    def grab(table_hbm, idx_hbm, out_hbm, idx_v0, idx_v1, rows_v0, rows_v1,
             tr_v0, tr_v1, g0, g1, s0, s1):
        wid = lax.axis_index("s") * _NC + lax.axis_index("c")
        base_u = wid * units_per_w
        idx_bufs = (idx_v0, idx_v1)
        row_bufs = (rows_v0, rows_v1)
        tr_bufs = (tr_v0, tr_v1)
        gsems = (g0, g1)
        ssems = (s0, s1)
        lane = lax.iota(jnp.int32, 16)

        def transpose_chunk(rows, tr):
            @plsc.parallel_loop(0, chunk // 16, 1, unroll=2)
            def _(blk):
                row_idx = blk * 16 + lane
                bt = blk >> 3
                bi0 = (blk & 7) * 16
                for dt in range(n_dt):
                    for di in range(8):
                        col_idx = jnp.full((16,), dt * 8 + di, jnp.int32)
                        v = plsc.load_gather(rows, [row_idx, col_idx])
                        tr[dt, bt, di, pl.ds(bi0, 16)] = v

        def out_slices(b, u):
            g = base_u + u
            s_idx = g // chunks_per_s
            bt0 = (g % chunks_per_s) * bt_per_chunk
            for dt in range(n_dt):
                yield (tr_bufs[b].at[dt],
                       out_hbm.at[s_idx, dt, pl.ds(bt0, bt_per_chunk)])

        def issue_gather(b, u):
            pltpu.sync_copy(
                idx_hbm.at[pl.ds((base_u + u) * chunk, chunk)], idx_bufs[b])
            pltpu.async_copy(table_hbm.at[idx_bufs[b]], row_bufs[b], gsems[b])

        def wait_gather(b):
            pltpu.make_async_copy(
                table_hbm.at[idx_bufs[b]], row_bufs[b], gsems[b]).wait()

        def issue_stores(b, u):
            for src, dst in out_slices(b, u):
                pltpu.async_copy(src, dst, ssems[b])

        def wait_stores(b, u):
            for src, dst in out_slices(b, u):
                pltpu.make_async_copy(src, dst, ssems[b]).wait()

        def process_unit(b, u, *, wait_store_u=None, next_u=None):
            if wait_store_u is not None:
                wait_stores(b, wait_store_u)
            wait_gather(b)
            transpose_chunk(row_bufs[b], tr_bufs[b])
            issue_stores(b, u)
            if next_u is not None:
                issue_gather(b, next_u)

        issue_gather(0, 0)
        issue_gather(1, 1)
        for b in (0, 1):
            process_unit(b, b, next_u=b + 2)

        def body(k, _):
            for b in (0, 1):
                u = 2 * k + b
                process_unit(b, u, wait_store_u=u - 2, next_u=u + 2)
            return ()

        lax.fori_loop(1, units_per_w // 2 - 1, body, ())

        for b in (0, 1):
            u = units_per_w - 2 + b
            process_unit(b, u, wait_store_u=u - 2)
        for b in (0, 1):
            wait_stores(b, units_per_w - 2 + b)

    return grab


def kernel(inputs, embedding_table):
    batch, seq = inputs.shape
    num_rows, dim = embedding_table.shape
    idx = inputs.T.reshape(-1).astype(jnp.int32)
    grab = _make_gather(num_rows, dim, batch, seq)
    out5 = grab(embedding_table, idx)
    return out5.transpose(2, 4, 0, 1, 3).reshape(batch, seq, dim)


# speedup vs baseline: 2.9318x; 1.4606x over previous
import functools

import jax
import jax.numpy as jnp
from jax import lax
from jax.experimental import pallas as pl
from jax.experimental.pallas import tpu as pltpu
from jax.experimental.pallas import tpu_sc as plsc

_info = plsc.get_sparse_core_info()
_NC = _info.num_cores
_NS = _info.num_subcores
_NW = _NC * _NS


@functools.lru_cache(maxsize=None)
def _make_gather(num_rows, dim, batch, seq):
    assert dim == 32 and batch % 512 == 0
    chunk = 512
    chunks_per_s = batch // chunk
    n_units = seq * chunks_per_s
    assert n_units % _NW == 0
    units_per_w = n_units // _NW
    assert units_per_w >= 4 and units_per_w % 2 == 0
    bt_per_chunk = chunk // 128
    n_dt = dim // 8

    mesh = plsc.VectorSubcoreMesh(core_axis_name="c", subcore_axis_name="s")

    @functools.partial(
        pl.kernel,
        out_type=jax.ShapeDtypeStruct(
            (seq, n_dt, batch // 128, 8, 128), jnp.float32),
        mesh=mesh,
        compiler_params=pltpu.CompilerParams(
            use_tc_tiling_on_sc=False, needs_layout_passes=False),
        scratch_types=[
            pltpu.VMEM((chunk,), jnp.int32),
            pltpu.VMEM((chunk,), jnp.int32),
            pltpu.VMEM((chunk, dim), jnp.float32),
            pltpu.VMEM((chunk, dim), jnp.float32),
            pltpu.VMEM((n_dt, bt_per_chunk, 8, 128), jnp.float32),
            pltpu.VMEM((n_dt, bt_per_chunk, 8, 128), jnp.float32),
            pltpu.SemaphoreType.DMA,
            pltpu.SemaphoreType.DMA,
            pltpu.SemaphoreType.DMA,
            pltpu.SemaphoreType.DMA,
        ],
    )
    def grab(table_hbm, idx_hbm, out_hbm, idx_v0, idx_v1, rows_v0, rows_v1,
             tr_v0, tr_v1, g0, g1, s0, s1):
        wid = lax.axis_index("s") * _NC + lax.axis_index("c")
        base_u = wid * units_per_w
        idx_bufs = (idx_v0, idx_v1)
        row_bufs = (rows_v0, rows_v1)
        tr_bufs = (tr_v0, tr_v1)
        gsems = (g0, g1)
        ssems = (s0, s1)
        lane = lax.iota(jnp.int32, 16)

        def transpose_chunk(rows, tr):
            @plsc.parallel_loop(0, chunk // 16, 1)
            def _(jb):
                row_idx = jb * 16 + lane
                bt_idx = jnp.broadcast_to(jb >> 3, (16,))
                bi_idx = (jb & 7) * 16 + lane
                for d0 in range(dim):
                    d = (d0 + lane) & (dim - 1)
                    v = plsc.load_gather(rows, [row_idx, d])
                    plsc.store_scatter(
                        tr, [d >> 3, bt_idx, d & 7, bi_idx], v)

        def out_slices(b, u):
            g = base_u + u
            s_idx = g // chunks_per_s
            bt0 = (g % chunks_per_s) * bt_per_chunk
            for dt in range(n_dt):
                yield (tr_bufs[b].at[dt],
                       out_hbm.at[s_idx, dt, pl.ds(bt0, bt_per_chunk)])

        def issue_gather(b, u):
            pltpu.sync_copy(
                idx_hbm.at[pl.ds((base_u + u) * chunk, chunk)], idx_bufs[b])
            pltpu.async_copy(table_hbm.at[idx_bufs[b]], row_bufs[b], gsems[b])

        def wait_gather(b):
            pltpu.make_async_copy(
                table_hbm.at[idx_bufs[b]], row_bufs[b], gsems[b]).wait()

        def issue_stores(b, u):
            for src, dst in out_slices(b, u):
                pltpu.async_copy(src, dst, ssems[b])

        def wait_stores(b, u):
            for src, dst in out_slices(b, u):
                pltpu.make_async_copy(src, dst, ssems[b]).wait()

        def process_unit(b, u, *, wait_store_u=None, next_u=None):
            if wait_store_u is not None:
                wait_stores(b, wait_store_u)
            wait_gather(b)
            transpose_chunk(row_bufs[b], tr_bufs[b])
            issue_stores(b, u)
            if next_u is not None:
                issue_gather(b, next_u)

        issue_gather(0, 0)
        issue_gather(1, 1)
        for b in (0, 1):
            process_unit(b, b, next_u=b + 2)

        def body(k, _):
            for b in (0, 1):
                u = 2 * k + b
                process_unit(b, u, wait_store_u=u - 2, next_u=u + 2)
            return ()

        lax.fori_loop(1, units_per_w // 2 - 1, body, ())

        for b in (0, 1):
            u = units_per_w - 2 + b
            process_unit(b, u, wait_store_u=u - 2)
        for b in (0, 1):
            wait_stores(b, units_per_w - 2 + b)

    return grab


def kernel(inputs, embedding_table):
    batch, seq = inputs.shape
    num_rows, dim = embedding_table.shape
    idx = inputs.T.reshape(-1).astype(jnp.int32)
    grab = _make_gather(num_rows, dim, batch, seq)
    out5 = grab(embedding_table, idx)
    return out5.transpose(2, 4, 0, 1, 3).reshape(batch, seq, dim)


# speedup vs baseline: 4.2997x; 1.4666x over previous
import functools

import jax
import jax.numpy as jnp
from jax import lax
from jax.experimental.layout import Format, Layout, with_layout_constraint
from jax.experimental import pallas as pl
from jax.experimental.pallas import tpu as pltpu
from jax.experimental.pallas import tpu_sc as plsc

_info = plsc.get_sparse_core_info()
_NC = _info.num_cores
_NS = _info.num_subcores
_NW = _NC * _NS


@functools.lru_cache(maxsize=None)
def _make_gather(num_rows, dim, batch, seq):
    assert dim == 32 and batch % 512 == 0
    chunk = 512
    chunks_per_s = batch // chunk
    n_units = seq * chunks_per_s
    assert n_units % _NW == 0
    units_per_w = n_units // _NW
    assert units_per_w >= 4 and units_per_w % 2 == 0
    bt_per_chunk = chunk // 128
    n_dt = dim // 8

    mesh = plsc.VectorSubcoreMesh(core_axis_name="c", subcore_axis_name="s")

    @functools.partial(
        pl.kernel,
        out_type=jax.ShapeDtypeStruct(
            (seq, n_dt, batch // 128, 8, 128), jnp.float32),
        mesh=mesh,
        compiler_params=pltpu.CompilerParams(
            use_tc_tiling_on_sc=False, needs_layout_passes=False),
        scratch_types=[
            pltpu.VMEM((chunk,), jnp.int32),
            pltpu.VMEM((chunk,), jnp.int32),
            pltpu.VMEM((chunk, dim), jnp.float32),
            pltpu.VMEM((chunk, dim), jnp.float32),
            pltpu.VMEM((n_dt, bt_per_chunk, 8, 128), jnp.float32),
            pltpu.VMEM((n_dt, bt_per_chunk, 8, 128), jnp.float32),
            pltpu.SemaphoreType.DMA,
            pltpu.SemaphoreType.DMA,
            pltpu.SemaphoreType.DMA,
            pltpu.SemaphoreType.DMA,
        ],
    )
    def grab(table_hbm, idx_hbm, out_hbm, idx_v0, idx_v1, rows_v0, rows_v1,
             tr_v0, tr_v1, g0, g1, s0, s1):
        wid = lax.axis_index("s") * _NC + lax.axis_index("c")
        base_u = wid * units_per_w
        idx_bufs = (idx_v0, idx_v1)
        row_bufs = (rows_v0, rows_v1)
        tr_bufs = (tr_v0, tr_v1)
        gsems = (g0, g1)
        ssems = (s0, s1)
        lane = lax.iota(jnp.int32, 16)

        def transpose_chunk(rows, tr):
            @plsc.parallel_loop(0, chunk // 16, 1)
            def _(jb):
                row_idx = jb * 16 + lane
                bt_idx = jnp.broadcast_to(jb >> 3, (16,))
                bi_idx = (jb & 7) * 16 + lane
                for d0 in range(dim):
                    d = (d0 + lane) & (dim - 1)
                    v = plsc.load_gather(rows, [row_idx, d])
                    plsc.store_scatter(
                        tr, [d >> 3, bt_idx, d & 7, bi_idx], v)

        def out_slices(b, u):
            g = base_u + u
            s_idx = g // chunks_per_s
            bt0 = (g % chunks_per_s) * bt_per_chunk
            for dt in range(n_dt):
                yield (tr_bufs[b].at[dt],
                       out_hbm.at[s_idx, dt, pl.ds(bt0, bt_per_chunk)])

        def issue_gather(b, u):
            pltpu.sync_copy(
                idx_hbm.at[pl.ds((base_u + u) * chunk, chunk)], idx_bufs[b])
            pltpu.async_copy(table_hbm.at[idx_bufs[b]], row_bufs[b], gsems[b])

        def wait_gather(b):
            pltpu.make_async_copy(
                table_hbm.at[idx_bufs[b]], row_bufs[b], gsems[b]).wait()

        def issue_stores(b, u):
            for src, dst in out_slices(b, u):
                pltpu.async_copy(src, dst, ssems[b])

        def wait_stores(b, u):
            for src, dst in out_slices(b, u):
                pltpu.make_async_copy(src, dst, ssems[b]).wait()

        def process_unit(b, u, *, wait_store_u=None, next_u=None):
            if wait_store_u is not None:
                wait_stores(b, wait_store_u)
            wait_gather(b)
            transpose_chunk(row_bufs[b], tr_bufs[b])
            issue_stores(b, u)
            if next_u is not None:
                issue_gather(b, next_u)

        issue_gather(0, 0)
        issue_gather(1, 1)
        for b in (0, 1):
            process_unit(b, b, next_u=b + 2)

        def body(k, _):
            for b in (0, 1):
                u = 2 * k + b
                process_unit(b, u, wait_store_u=u - 2, next_u=u + 2)
            return ()

        lax.fori_loop(1, units_per_w // 2 - 1, body, ())

        for b in (0, 1):
            u = units_per_w - 2 + b
            process_unit(b, u, wait_store_u=u - 2)
        for b in (0, 1):
            wait_stores(b, units_per_w - 2 + b)

    return grab


def kernel(inputs, embedding_table):
    batch, seq = inputs.shape
    num_rows, dim = embedding_table.shape
    idx = inputs.T.reshape(-1).astype(jnp.int32)
    table_rm = with_layout_constraint(
        embedding_table,
        Layout(major_to_minor=(0, 1), tiling=((8,), (1024,))))
    grab = _make_gather(num_rows, dim, batch, seq)
    out5 = grab(table_rm, idx)
    return out5.transpose(2, 4, 0, 1, 3).reshape(batch, seq, dim)


# speedup vs baseline: 6.4673x; 1.5041x over previous
import functools

import jax
import jax.numpy as jnp
from jax import lax
from jax.experimental import pallas as pl
from jax.experimental.pallas import tpu as pltpu
from jax.experimental.pallas import tpu_sc as plsc

_info = plsc.get_sparse_core_info()
_NC = _info.num_cores
_NS = _info.num_subcores
_NW = _NC * _NS


@functools.lru_cache(maxsize=None)
def _make_detile(num_rows, dim):
    assert dim == 32
    cu = 512
    n_full = num_rows // cu
    rem = num_rows - n_full * cu
    k_steady = n_full // _NW
    n_extra = n_full - k_steady * _NW
    assert k_steady >= 4
    assert n_extra + (1 if rem else 0) <= _NW

    mesh = plsc.VectorSubcoreMesh(core_axis_name="c", subcore_axis_name="s")

    @functools.partial(
        pl.kernel,
        out_type=jax.ShapeDtypeStruct((num_rows * dim,), jnp.float32),
        mesh=mesh,
        compiler_params=pltpu.CompilerParams(
            use_tc_tiling_on_sc=True, needs_layout_passes=False),
        scratch_types=[
            pltpu.VMEM((dim, cu), jnp.float32),
            pltpu.VMEM((dim, cu), jnp.float32),
            pltpu.VMEM((cu * dim,), jnp.float32),
            pltpu.VMEM((cu * dim,), jnp.float32),
            pltpu.VMEM((dim, max(rem, 16)), jnp.float32),
            pltpu.VMEM((max(rem, 16) * dim,), jnp.float32),
            pltpu.SemaphoreType.DMA,
            pltpu.SemaphoreType.DMA,
            pltpu.SemaphoreType.DMA,
            pltpu.SemaphoreType.DMA,
        ],
    )
    def detile(t32_hbm, out_hbm, in_v0, in_v1, out_v0, out_v1,
               in_vr, out_vr, g0, g1, s0, s1):
        wid = lax.axis_index("s") * _NC + lax.axis_index("c")
        in_bufs = (in_v0, in_v1)
        out_bufs = (out_v0, out_v1)
        gsems = (g0, g1)
        ssems = (s0, s1)
        lane = lax.iota(jnp.int32, 16)

        def transpose_block(in_ref, out_ref, width):
            @plsc.parallel_loop(0, width // 16, 1)
            def _(cb):
                cvec = cb * 16 + lane
                for d0 in range(dim):
                    d = (d0 + lane) & (dim - 1)
                    v = plsc.load_gather(in_ref, [d, cvec])
                    plsc.store_scatter(out_ref, [cvec * dim + d], v)

        def c0_of(k):
            return (k * _NW + wid) * cu

        def issue_in(b, k):
            pltpu.async_copy(t32_hbm.at[:, pl.ds(c0_of(k), cu)], in_bufs[b],
                             gsems[b])

        def wait_in(b, k):
            pltpu.make_async_copy(t32_hbm.at[:, pl.ds(c0_of(k), cu)],
                                  in_bufs[b], gsems[b]).wait()

        def issue_out(b, k):
            pltpu.async_copy(out_bufs[b],
                             out_hbm.at[pl.ds(c0_of(k) * dim, cu * dim)],
                             ssems[b])

        def wait_out(b, k):
            pltpu.make_async_copy(out_bufs[b],
                                  out_hbm.at[pl.ds(c0_of(k) * dim, cu * dim)],
                                  ssems[b]).wait()

        def process(b, k, *, wait_store_k=None, next_k=None):
            if wait_store_k is not None:
                wait_out(b, wait_store_k)
            wait_in(b, k)
            transpose_block(in_bufs[b], out_bufs[b], cu)
            issue_out(b, k)
            if next_k is not None:
                issue_in(b, next_k)

        issue_in(0, 0)
        issue_in(1, 1)
        process(0, 0, next_k=2)
        process(1, 1, next_k=3)

        j_hi = (k_steady - 2) // 2
        def body(j, _):
            for b in (0, 1):
                k = 2 * j + b
                process(b, k, wait_store_k=k - 2, next_k=k + 2)
            return ()

        lax.fori_loop(1, j_hi, body, ())
        for k in range(2 * j_hi, k_steady):
            b = k % 2
            nk = k + 2 if k + 2 < k_steady else None
            process(b, k, wait_store_k=k - 2, next_k=nk)
        for k in (k_steady - 2, k_steady - 1):
            wait_out(k % 2, k)

        for t in range(n_extra):
            @pl.when(wid == t)
            def _():
                c0 = (k_steady * _NW + t) * cu
                pltpu.sync_copy(t32_hbm.at[:, pl.ds(c0, cu)], in_v0)
                transpose_block(in_v0, out_v0, cu)
                pltpu.sync_copy(out_v0,
                                out_hbm.at[pl.ds(c0 * dim, cu * dim)])

        if rem:
            @pl.when(wid == n_extra)
            def _():
                c0 = n_full * cu
                pltpu.sync_copy(t32_hbm.at[:, pl.ds(c0, rem)], in_vr)
                transpose_block(in_vr, out_vr, rem)
                pltpu.sync_copy(out_vr,
                                out_hbm.at[pl.ds(c0 * dim, rem * dim)])

    return detile


@functools.lru_cache(maxsize=None)
def _make_gather(num_rows, dim, batch, seq):
    assert dim == 32 and batch % 512 == 0
    chunk = 512
    chunks_per_s = batch // chunk
    n_units = seq * chunks_per_s
    assert n_units % _NW == 0
    units_per_w = n_units // _NW
    assert units_per_w >= 4 and units_per_w % 2 == 0
    bt_per_chunk = chunk // 128
    n_dt = dim // 8

    mesh = plsc.VectorSubcoreMesh(core_axis_name="c", subcore_axis_name="s")

    @functools.partial(
        pl.kernel,
        out_type=jax.ShapeDtypeStruct(
            (seq, n_dt, batch // 128, 8, 128), jnp.float32),
        mesh=mesh,
        compiler_params=pltpu.CompilerParams(
            use_tc_tiling_on_sc=False, needs_layout_passes=False),
        scratch_types=[
            pltpu.VMEM((chunk,), jnp.int32),
            pltpu.VMEM((chunk,), jnp.int32),
            pltpu.VMEM((chunk, dim), jnp.float32),
            pltpu.VMEM((chunk, dim), jnp.float32),
            pltpu.VMEM((n_dt, bt_per_chunk, 8, 128), jnp.float32),
            pltpu.VMEM((n_dt, bt_per_chunk, 8, 128), jnp.float32),
            pltpu.SemaphoreType.DMA,
            pltpu.SemaphoreType.DMA,
            pltpu.SemaphoreType.DMA,
            pltpu.SemaphoreType.DMA,
        ],
    )
    def grab(table_hbm, idx_hbm, out_hbm, idx_v0, idx_v1, rows_v0, rows_v1,
             tr_v0, tr_v1, g0, g1, s0, s1):
        wid = lax.axis_index("s") * _NC + lax.axis_index("c")
        base_u = wid * units_per_w
        idx_bufs = (idx_v0, idx_v1)
        row_bufs = (rows_v0, rows_v1)
        tr_bufs = (tr_v0, tr_v1)
        gsems = (g0, g1)
        ssems = (s0, s1)
        lane = lax.iota(jnp.int32, 16)

        def transpose_chunk(rows, tr):
            @plsc.parallel_loop(0, chunk // 16, 1)
            def _(jb):
                row_idx = jb * 16 + lane
                bt_idx = jnp.broadcast_to(jb >> 3, (16,))
                bi_idx = (jb & 7) * 16 + lane
                for d0 in range(dim):
                    d = (d0 + lane) & (dim - 1)
                    v = plsc.load_gather(rows, [row_idx, d])
                    plsc.store_scatter(
                        tr, [d >> 3, bt_idx, d & 7, bi_idx], v)

        def out_slices(b, u):
            g = base_u + u
            s_idx = g // chunks_per_s
            bt0 = (g % chunks_per_s) * bt_per_chunk
            for dt in range(n_dt):
                yield (tr_bufs[b].at[dt],
                       out_hbm.at[s_idx, dt, pl.ds(bt0, bt_per_chunk)])

        def issue_gather(b, u):
            pltpu.sync_copy(
                idx_hbm.at[pl.ds((base_u + u) * chunk, chunk)], idx_bufs[b])
            pltpu.async_copy(table_hbm.at[idx_bufs[b]], row_bufs[b], gsems[b])

        def wait_gather(b):
            pltpu.make_async_copy(
                table_hbm.at[idx_bufs[b]], row_bufs[b], gsems[b]).wait()

        def issue_stores(b, u):
            for src, dst in out_slices(b, u):
                pltpu.async_copy(src, dst, ssems[b])

        def wait_stores(b, u):
            for src, dst in out_slices(b, u):
                pltpu.make_async_copy(src, dst, ssems[b]).wait()

        def process_unit(b, u, *, wait_store_u=None, next_u=None):
            if wait_store_u is not None:
                wait_stores(b, wait_store_u)
            wait_gather(b)
            transpose_chunk(row_bufs[b], tr_bufs[b])
            issue_stores(b, u)
            if next_u is not None:
                issue_gather(b, next_u)

        issue_gather(0, 0)
        issue_gather(1, 1)
        for b in (0, 1):
            process_unit(b, b, next_u=b + 2)

        def body(k, _):
            for b in (0, 1):
                u = 2 * k + b
                process_unit(b, u, wait_store_u=u - 2, next_u=u + 2)
            return ()

        lax.fori_loop(1, units_per_w // 2 - 1, body, ())

        for b in (0, 1):
            u = units_per_w - 2 + b
            process_unit(b, u, wait_store_u=u - 2)
        for b in (0, 1):
            wait_stores(b, units_per_w - 2 + b)

    return grab


def kernel(inputs, embedding_table):
    batch, seq = inputs.shape
    num_rows, dim = embedding_table.shape
    idx = inputs.T.reshape(-1).astype(jnp.int32)
    flat = _make_detile(num_rows, dim)(embedding_table.T)
    table_rm = flat.reshape(num_rows, dim)
    out5 = _make_gather(num_rows, dim, batch, seq)(table_rm, idx)
    return out5.transpose(2, 4, 0, 1, 3).reshape(batch, seq, dim)


# speedup vs baseline: 7.1114x; 1.0996x over previous
import functools

import jax
import jax.numpy as jnp
from jax import lax
from jax.experimental import pallas as pl
from jax.experimental.pallas import tpu as pltpu
from jax.experimental.pallas import tpu_sc as plsc

_info = plsc.get_sparse_core_info()
_NC = _info.num_cores
_NS = _info.num_subcores
_NW = _NC * _NS


@functools.lru_cache(maxsize=None)
def _make_detile(num_rows, dim):
    assert dim == 32
    cu = 512
    n_full = num_rows // cu
    rem = num_rows - n_full * cu
    k_steady = n_full // _NW
    n_extra = n_full - k_steady * _NW
    assert k_steady >= 4
    assert n_extra + (1 if rem else 0) <= _NW

    mesh = plsc.VectorSubcoreMesh(core_axis_name="c", subcore_axis_name="s")

    @functools.partial(
        pl.kernel,
        out_type=jax.ShapeDtypeStruct((num_rows * dim,), jnp.float32),
        mesh=mesh,
        compiler_params=pltpu.CompilerParams(
            use_tc_tiling_on_sc=True, needs_layout_passes=False),
        scratch_types=[
            pltpu.VMEM((dim, cu), jnp.float32),
            pltpu.VMEM((dim, cu), jnp.float32),
            pltpu.VMEM((cu * dim,), jnp.float32),
            pltpu.VMEM((cu * dim,), jnp.float32),
            pltpu.VMEM((dim, max(rem, 16)), jnp.float32),
            pltpu.VMEM((max(rem, 16) * dim,), jnp.float32),
            pltpu.SemaphoreType.DMA,
            pltpu.SemaphoreType.DMA,
            pltpu.SemaphoreType.DMA,
            pltpu.SemaphoreType.DMA,
        ],
    )
    def detile(t32_hbm, out_hbm, in_v0, in_v1, out_v0, out_v1,
               in_vr, out_vr, g0, g1, s0, s1):
        wid = lax.axis_index("s") * _NC + lax.axis_index("c")
        in_bufs = (in_v0, in_v1)
        out_bufs = (out_v0, out_v1)
        gsems = (g0, g1)
        ssems = (s0, s1)
        lane = lax.iota(jnp.int32, 16)

        def transpose_block(in_ref, out_ref, width):
            @plsc.parallel_loop(0, width // 16, 1, unroll=2)
            def _(cb):
                cvec = cb * 16 + lane
                for d0 in range(dim):
                    d = (d0 + lane) & (dim - 1)
                    v = plsc.load_gather(in_ref, [d, cvec])
                    plsc.store_scatter(out_ref, [cvec * dim + d], v)

        def c0_of(k):
            return (k * _NW + wid) * cu

        def issue_in(b, k):
            pltpu.async_copy(t32_hbm.at[:, pl.ds(c0_of(k), cu)], in_bufs[b],
                             gsems[b])

        def wait_in(b, k):
            pltpu.make_async_copy(t32_hbm.at[:, pl.ds(c0_of(k), cu)],
                                  in_bufs[b], gsems[b]).wait()

        def issue_out(b, k):
            pltpu.async_copy(out_bufs[b],
                             out_hbm.at[pl.ds(c0_of(k) * dim, cu * dim)],
                             ssems[b])

        def wait_out(b, k):
            pltpu.make_async_copy(out_bufs[b],
                                  out_hbm.at[pl.ds(c0_of(k) * dim, cu * dim)],
                                  ssems[b]).wait()

        def process(b, k, *, wait_store_k=None, next_k=None):
            if wait_store_k is not None:
                wait_out(b, wait_store_k)
            wait_in(b, k)
            transpose_block(in_bufs[b], out_bufs[b], cu)
            issue_out(b, k)
            if next_k is not None:
                issue_in(b, next_k)

        issue_in(0, 0)
        issue_in(1, 1)
        process(0, 0, next_k=2)
        process(1, 1, next_k=3)

        j_hi = (k_steady - 2) // 2
        def body(j, _):
            for b in (0, 1):
                k = 2 * j + b
                process(b, k, wait_store_k=k - 2, next_k=k + 2)
            return ()

        lax.fori_loop(1, j_hi, body, ())
        for k in range(2 * j_hi, k_steady):
            b = k % 2
            nk = k + 2 if k + 2 < k_steady else None
            process(b, k, wait_store_k=k - 2, next_k=nk)
        for k in (k_steady - 2, k_steady - 1):
            wait_out(k % 2, k)

        for t in range(n_extra):
            @pl.when(wid == t)
            def _():
                c0 = (k_steady * _NW + t) * cu
                pltpu.sync_copy(t32_hbm.at[:, pl.ds(c0, cu)], in_v0)
                transpose_block(in_v0, out_v0, cu)
                pltpu.sync_copy(out_v0,
                                out_hbm.at[pl.ds(c0 * dim, cu * dim)])

        if rem:
            @pl.when(wid == n_extra)
            def _():
                c0 = n_full * cu
                pltpu.sync_copy(t32_hbm.at[:, pl.ds(c0, rem)], in_vr)
                transpose_block(in_vr, out_vr, rem)
                pltpu.sync_copy(out_vr,
                                out_hbm.at[pl.ds(c0 * dim, rem * dim)])

    return detile


@functools.lru_cache(maxsize=None)
def _make_gather(num_rows, dim, batch, seq):
    assert dim == 32 and batch % 512 == 0
    chunk = 512
    chunks_per_s = batch // chunk
    n_units = seq * chunks_per_s
    assert n_units % _NW == 0
    units_per_w = n_units // _NW
    assert units_per_w >= 4 and units_per_w % 2 == 0
    bt_per_chunk = chunk // 128
    n_dt = dim // 8

    mesh = plsc.VectorSubcoreMesh(core_axis_name="c", subcore_axis_name="s")

    @functools.partial(
        pl.kernel,
        out_type=jax.ShapeDtypeStruct(
            (seq, n_dt, batch // 128, 8, 128), jnp.float32),
        mesh=mesh,
        compiler_params=pltpu.CompilerParams(
            use_tc_tiling_on_sc=False, needs_layout_passes=False),
        scratch_types=[
            pltpu.VMEM((chunk,), jnp.int32),
            pltpu.VMEM((chunk,), jnp.int32),
            pltpu.VMEM((chunk, dim), jnp.float32),
            pltpu.VMEM((chunk, dim), jnp.float32),
            pltpu.VMEM((n_dt, bt_per_chunk, 8, 128), jnp.float32),
            pltpu.VMEM((n_dt, bt_per_chunk, 8, 128), jnp.float32),
            pltpu.SemaphoreType.DMA,
            pltpu.SemaphoreType.DMA,
            pltpu.SemaphoreType.DMA,
            pltpu.SemaphoreType.DMA,
        ],
    )
    def grab(table_hbm, idx_hbm, out_hbm, idx_v0, idx_v1, rows_v0, rows_v1,
             tr_v0, tr_v1, g0, g1, s0, s1):
        wid = lax.axis_index("s") * _NC + lax.axis_index("c")
        base_u = wid * units_per_w
        idx_bufs = (idx_v0, idx_v1)
        row_bufs = (rows_v0, rows_v1)
        tr_bufs = (tr_v0, tr_v1)
        gsems = (g0, g1)
        ssems = (s0, s1)
        lane = lax.iota(jnp.int32, 16)

        def transpose_chunk(rows, tr):
            @plsc.parallel_loop(0, chunk // 16, 1, unroll=2)
            def _(jb):
                row_idx = jb * 16 + lane
                bt_idx = jnp.broadcast_to(jb >> 3, (16,))
                bi_idx = (jb & 7) * 16 + lane
                for d0 in range(dim):
                    d = (d0 + lane) & (dim - 1)
                    v = plsc.load_gather(rows, [row_idx, d])
                    plsc.store_scatter(
                        tr, [d >> 3, bt_idx, d & 7, bi_idx], v)

        def out_slices(b, u):
            g = base_u + u
            s_idx = g // chunks_per_s
            bt0 = (g % chunks_per_s) * bt_per_chunk
            for dt in range(n_dt):
                yield (tr_bufs[b].at[dt],
                       out_hbm.at[s_idx, dt, pl.ds(bt0, bt_per_chunk)])

        def issue_gather(b, u):
            pltpu.sync_copy(
                idx_hbm.at[pl.ds((base_u + u) * chunk, chunk)], idx_bufs[b])
            pltpu.async_copy(table_hbm.at[idx_bufs[b]], row_bufs[b], gsems[b])

        def wait_gather(b):
            pltpu.make_async_copy(
                table_hbm.at[idx_bufs[b]], row_bufs[b], gsems[b]).wait()

        def issue_stores(b, u):
            for src, dst in out_slices(b, u):
                pltpu.async_copy(src, dst, ssems[b])

        def wait_stores(b, u):
            for src, dst in out_slices(b, u):
                pltpu.make_async_copy(src, dst, ssems[b]).wait()

        def process_unit(b, u, *, wait_store_u=None, next_u=None):
            if wait_store_u is not None:
                wait_stores(b, wait_store_u)
            wait_gather(b)
            transpose_chunk(row_bufs[b], tr_bufs[b])
            issue_stores(b, u)
            if next_u is not None:
                issue_gather(b, next_u)

        issue_gather(0, 0)
        issue_gather(1, 1)
        for b in (0, 1):
            process_unit(b, b, next_u=b + 2)

        def body(k, _):
            for b in (0, 1):
                u = 2 * k + b
                process_unit(b, u, wait_store_u=u - 2, next_u=u + 2)
            return ()

        lax.fori_loop(1, units_per_w // 2 - 1, body, ())

        for b in (0, 1):
            u = units_per_w - 2 + b
            process_unit(b, u, wait_store_u=u - 2)
        for b in (0, 1):
            wait_stores(b, units_per_w - 2 + b)

    return grab


def kernel(inputs, embedding_table):
    batch, seq = inputs.shape
    num_rows, dim = embedding_table.shape
    idx = inputs.T.reshape(-1).astype(jnp.int32)
    flat = _make_detile(num_rows, dim)(embedding_table.T)
    table_rm = flat.reshape(num_rows, dim)
    out5 = _make_gather(num_rows, dim, batch, seq)(table_rm, idx)
    return out5.transpose(2, 4, 0, 1, 3).reshape(batch, seq, dim)
